# baseline (speedup 1.0000x reference)
import functools

import jax
import jax.numpy as jnp
from jax import lax
from jax.experimental import pallas as pl
from jax.experimental.pallas import tpu as pltpu
from jax.experimental.pallas import tpu_sc as plsc

NC = 2
NS = 16
NW = NC * NS

L = 16
G = 128
CHUNK = 512
GPC = CHUNK // G
VPG = G // L


def _gather_call(n_tokens, dim, n_base, n_spec, bpw):
    ng = bpw // G
    nch = bpw // CHUNK

    mesh = plsc.VectorSubcoreMesh(core_axis_name="c", subcore_axis_name="s")

    @functools.partial(
        pl.kernel,
        out_type=jax.ShapeDtypeStruct((n_tokens, dim), jnp.float32),
        mesh=mesh,
        compiler_params=pltpu.CompilerParams(
            use_tc_tiling_on_sc=False, needs_layout_passes=False),
        scratch_types=[
            pltpu.VMEM((ng, G), jnp.int32),
            pltpu.VMEM((n_spec, dim), jnp.float32),
            pltpu.VMEM((CHUNK, dim), jnp.float32),
            pltpu.VMEM((GPC, G), jnp.int32),
            pltpu.SemaphoreType.DMA,
        ],
    )
    def gather_kernel(ids_hbm, base_hbm, spec_hbm, out_hbm,
                      idx_v, spec_v, rows_v, bidx_v, gsem):
        wid = lax.axis_index("s") * NC + lax.axis_index("c")
        base = wid * bpw
        pltpu.sync_copy(ids_hbm.at[wid], idx_v)
        pltpu.sync_copy(spec_hbm, spec_v)

        def chunk_body(c, _):
            row0 = c * GPC
            for g in range(GPC):
                for s in range(VPG):
                    ids16 = idx_v[row0 + g, pl.ds(s * L, L)]
                    bidx_v[g, pl.ds(s * L, L)] = jnp.where(
                        ids16 < n_base, ids16, 0)
            descs = []
            for g in range(GPC):
                descs.append(pltpu.async_copy(
                    base_hbm.at[bidx_v.at[g]],
                    rows_v.at[pl.ds(g * G, G)],
                    gsem,
                ))
            for d in descs:
                d.wait()
            for g in range(GPC):
                for s in range(VPG):
                    ids16 = idx_v[row0 + g, pl.ds(s * L, L)]
                    srow = ids16 - n_base
                    smask = srow >= 0

                    @pl.when(jnp.max(srow) >= 0)
                    def _(srow=srow, smask=smask, g=g, s=s):
                        srow_c = jnp.clip(srow, 0, n_spec - 1)
                        rowi = lax.iota(jnp.int32, L) + (g * G + s * L)

                        def dim_body(d, _):
                            col = jnp.full((L,), d, jnp.int32)
                            x = plsc.load_gather(spec_v, [srow_c, col])
                            plsc.store_scatter(
                                rows_v, [rowi, col], x, mask=smask)
                            return 0

                        lax.fori_loop(0, dim, dim_body, 0)
            pltpu.sync_copy(
                rows_v, out_hbm.at[pl.ds(base + c * CHUNK, CHUNK)])
            return 0

        lax.fori_loop(0, nch, chunk_body, 0)

    return gather_kernel


def kernel(input_ids, base_table, special_A, special_B, lookup_A, lookup_B):
    del lookup_A, lookup_B
    dim = base_table.shape[1]
    n_base = base_table.shape[0]
    spec = jnp.concatenate([special_A, special_B], axis=0)
    n_tokens = input_ids.shape[0] * input_ids.shape[1]
    bpw = n_tokens // NW
    ids3d = input_ids.reshape(NW, bpw // G, G)
    out = _gather_call(n_tokens, dim, n_base, spec.shape[0], bpw)(
        ids3d, base_table, spec)
    return out.reshape(input_ids.shape + (dim,))


# speedup vs baseline: 1.1131x; 1.1131x over previous
import functools

import jax
import jax.numpy as jnp
from jax import lax
from jax.experimental import pallas as pl
from jax.experimental.pallas import tpu as pltpu
from jax.experimental.pallas import tpu_sc as plsc

NC = 2
NS = 16
NW = NC * NS

G = 128
CHUNK = 512
GPC = CHUNK // G
PADW = 128


def _gather_call(n_tokens, dim, bpw):
    ng = bpw // G
    nch = bpw // CHUNK

    mesh = plsc.VectorSubcoreMesh(core_axis_name="c", subcore_axis_name="s")

    @functools.partial(
        pl.kernel,
        out_type=jax.ShapeDtypeStruct((n_tokens, dim), jnp.float32),
        mesh=mesh,
        compiler_params=pltpu.CompilerParams(use_tc_tiling_on_sc=False),
        scratch_types=[
            pltpu.VMEM((ng, G), jnp.int32),
            pltpu.VMEM((CHUNK, PADW), jnp.float32),
            pltpu.SemaphoreType.DMA,
        ],
    )
    def gather_kernel(ids_hbm, table_hbm, out_hbm, idx_v, rows_v, gsem):
        wid = lax.axis_index("s") * NC + lax.axis_index("c")
        base = wid * bpw
        pltpu.sync_copy(ids_hbm.at[wid], idx_v)

        def chunk_body(c, _):
            descs = []
            for g in range(GPC):
                descs.append(pltpu.async_copy(
                    table_hbm.at[idx_v.at[c * GPC + g]],
                    rows_v.at[pl.ds(g * G, G)],
                    gsem,
                ))
            for d in descs:
                d.wait()
            pltpu.sync_copy(
                rows_v.at[:, pl.ds(0, dim)],
                out_hbm.at[pl.ds(base + c * CHUNK, CHUNK)],
            )
            return 0

        lax.fori_loop(0, nch, chunk_body, 0)

    return gather_kernel


def kernel(input_ids, base_table, special_A, special_B, lookup_A, lookup_B):
    del lookup_A, lookup_B
    dim = base_table.shape[1]
    table = jnp.concatenate([base_table, special_A, special_B], axis=0)
    table128 = jnp.pad(table, ((0, 0), (0, PADW - dim)))
    n_tokens = input_ids.shape[0] * input_ids.shape[1]
    bpw = n_tokens // NW
    ids3d = input_ids.reshape(NW, bpw // G, G)
    out = _gather_call(n_tokens, dim, bpw)(ids3d, table128)
    return out.reshape(input_ids.shape + (dim,))


# speedup vs baseline: 1.2263x; 1.1017x over previous
import functools

import jax
import jax.numpy as jnp
from jax import lax
from jax.experimental import pallas as pl
from jax.experimental.pallas import tpu as pltpu
from jax.experimental.pallas import tpu_sc as plsc

NC = 2
NS = 16
NW = NC * NS

CB = 2


def _gather_call(batch, seq, dim):
    bpw = batch // NW
    nch = bpw // CB
    s0 = (seq // 2 + 7) // 8 * 8

    mesh = plsc.VectorSubcoreMesh(core_axis_name="c", subcore_axis_name="s")

    @functools.partial(
        pl.kernel,
        out_type=jax.ShapeDtypeStruct((batch, seq, dim), jnp.float32),
        mesh=mesh,
        compiler_params=pltpu.CompilerParams(use_tc_tiling_on_sc=False),
        scratch_types=[
            pltpu.VMEM((bpw, seq), jnp.int32),
            pltpu.VMEM((CB, seq, dim), jnp.float32),
            pltpu.SemaphoreType.DMA,
        ],
    )
    def gather_kernel(ids_hbm, table_hbm, out_hbm, idx_v, rows_v, gsem):
        wid = lax.axis_index("s") * NC + lax.axis_index("c")
        b0 = wid * bpw
        pltpu.sync_copy(ids_hbm.at[pl.ds(b0, bpw)], idx_v)

        def chunk_body(c, _):
            descs = []
            for br in range(CB):
                r = c * CB + br
                descs.append(pltpu.async_copy(
                    table_hbm.at[idx_v.at[r, pl.ds(0, s0)]],
                    rows_v.at[br].at[pl.ds(0, s0)],
                    gsem,
                ))
                descs.append(pltpu.async_copy(
                    table_hbm.at[idx_v.at[r, pl.ds(s0, seq - s0)]],
                    rows_v.at[br].at[pl.ds(s0, seq - s0)],
                    gsem,
                ))
            for d in descs:
                d.wait()
            pltpu.sync_copy(rows_v, out_hbm.at[pl.ds(b0 + c * CB, CB)])
            return 0

        lax.fori_loop(0, nch, chunk_body, 0)

    return gather_kernel


def kernel(input_ids, base_table, special_A, special_B, lookup_A, lookup_B):
    del lookup_A, lookup_B
    batch, seq = input_ids.shape
    dim = base_table.shape[1]
    table = jnp.concatenate([base_table, special_A, special_B], axis=0)
    return _gather_call(batch, seq, dim)(input_ids, table)


# speedup vs baseline: 1.2818x; 1.0453x over previous
import functools

import jax
import jax.numpy as jnp
from jax import lax
from jax.experimental import pallas as pl
from jax.experimental.pallas import tpu as pltpu
from jax.experimental.pallas import tpu_sc as plsc

NC = 2
NS = 16
NW = NC * NS

CB = 2
NBUF = 2


def _gather_call(batch, seq, dim):
    bpw = batch // NW
    nch = bpw // CB
    s0 = (seq // 2 + 7) // 8 * 8

    mesh = plsc.VectorSubcoreMesh(core_axis_name="c", subcore_axis_name="s")

    @functools.partial(
        pl.kernel,
        out_type=jax.ShapeDtypeStruct((batch, seq, dim), jnp.float32),
        mesh=mesh,
        compiler_params=pltpu.CompilerParams(use_tc_tiling_on_sc=False),
        scratch_types=[
            pltpu.VMEM((bpw, seq), jnp.int32),
            pltpu.VMEM((NBUF, CB, seq, dim), jnp.float32),
            pltpu.SemaphoreType.DMA,
            pltpu.SemaphoreType.DMA,
        ],
    )
    def gather_kernel(ids_hbm, table_hbm, out_hbm, idx_v, rows_v, gsem, osem):
        wid = lax.axis_index("s") * NC + lax.axis_index("c")
        b0 = wid * bpw
        pltpu.sync_copy(ids_hbm.at[pl.ds(b0, bpw)], idx_v)

        def chunk_body(c, _):
            for b in range(NBUF):
                @pl.when(lax.rem(c, NBUF) == b)
                def _(b=b):
                    @pl.when(c >= NBUF)
                    def _():
                        pltpu.make_async_copy(
                            rows_v.at[b],
                            out_hbm.at[pl.ds(b0 + (c - NBUF) * CB, CB)],
                            osem,
                        ).wait()
                    descs = []
                    for br in range(CB):
                        r = c * CB + br
                        descs.append(pltpu.async_copy(
                            table_hbm.at[idx_v.at[r, pl.ds(0, s0)]],
                            rows_v.at[b, br].at[pl.ds(0, s0)],
                            gsem,
                        ))
                        descs.append(pltpu.async_copy(
                            table_hbm.at[idx_v.at[r, pl.ds(s0, seq - s0)]],
                            rows_v.at[b, br].at[pl.ds(s0, seq - s0)],
                            gsem,
                        ))
                    for d in descs:
                        d.wait()
                    pltpu.async_copy(
                        rows_v.at[b],
                        out_hbm.at[pl.ds(b0 + c * CB, CB)],
                        osem,
                    )
            return 0

        lax.fori_loop(0, nch, chunk_body, 0)
        for b in range(NBUF):
            c = nch - NBUF + b
            pltpu.make_async_copy(
                rows_v.at[c % NBUF],
                out_hbm.at[pl.ds(b0 + c * CB, CB)],
                osem,
            ).wait()

    return gather_kernel


def kernel(input_ids, base_table, special_A, special_B, lookup_A, lookup_B):
    del lookup_A, lookup_B
    batch, seq = input_ids.shape
    dim = base_table.shape[1]
    table = jnp.concatenate([base_table, special_A, special_B], axis=0)
    return _gather_call(batch, seq, dim)(input_ids, table)


# speedup vs baseline: 1.2907x; 1.0070x over previous
import functools

import jax
import jax.numpy as jnp
from jax import lax
from jax.experimental import pallas as pl
from jax.experimental.pallas import tpu as pltpu
from jax.experimental.pallas import tpu_sc as plsc

NC = 2
NS = 16
NW = NC * NS

CB = 4
NBUF = 2


def _gather_call(batch, seq, dim):
    bpw = batch // NW
    nch = bpw // CB
    s0 = (seq // 2 + 7) // 8 * 8

    mesh = plsc.VectorSubcoreMesh(core_axis_name="c", subcore_axis_name="s")

    @functools.partial(
        pl.kernel,
        out_type=jax.ShapeDtypeStruct((batch, seq, dim), jnp.float32),
        mesh=mesh,
        compiler_params=pltpu.CompilerParams(use_tc_tiling_on_sc=False),
        scratch_types=[
            pltpu.VMEM((bpw, seq), jnp.int32),
            pltpu.VMEM((NBUF, CB, seq, dim), jnp.float32),
            pltpu.SemaphoreType.DMA,
            pltpu.SemaphoreType.DMA,
        ],
    )
    def gather_kernel(ids_hbm, table_hbm, out_hbm, idx_v, rows_v, gsem, osem):
        wid = lax.axis_index("s") * NC + lax.axis_index("c")
        b0 = wid * bpw
        pltpu.sync_copy(ids_hbm.at[pl.ds(b0, bpw)], idx_v)

        def chunk_body(c, _):
            for b in range(NBUF):
                @pl.when(lax.rem(c, NBUF) == b)
                def _(b=b):
                    @pl.when(c >= NBUF)
                    def _():
                        pltpu.make_async_copy(
                            rows_v.at[b],
                            out_hbm.at[pl.ds(b0 + (c - NBUF) * CB, CB)],
                            osem,
                        ).wait()
                    descs = []
                    for br in range(CB):
                        r = c * CB + br
                        descs.append(pltpu.async_copy(
                            table_hbm.at[idx_v.at[r, pl.ds(0, s0)]],
                            rows_v.at[b, br].at[pl.ds(0, s0)],
                            gsem,
                        ))
                        descs.append(pltpu.async_copy(
                            table_hbm.at[idx_v.at[r, pl.ds(s0, seq - s0)]],
                            rows_v.at[b, br].at[pl.ds(s0, seq - s0)],
                            gsem,
                        ))
                    for d in descs:
                        d.wait()
                    pltpu.async_copy(
                        rows_v.at[b],
                        out_hbm.at[pl.ds(b0 + c * CB, CB)],
                        osem,
                    )
            return 0

        lax.fori_loop(0, nch, chunk_body, 0)
        for b in range(NBUF):
            c = nch - NBUF + b
            pltpu.make_async_copy(
                rows_v.at[c % NBUF],
                out_hbm.at[pl.ds(b0 + c * CB, CB)],
                osem,
            ).wait()

    return gather_kernel


def kernel(input_ids, base_table, special_A, special_B, lookup_A, lookup_B):
    del lookup_A, lookup_B
    batch, seq = input_ids.shape
    dim = base_table.shape[1]
    table = jnp.concatenate([base_table, special_A, special_B], axis=0)
    return _gather_call(batch, seq, dim)(input_ids, table)


# speedup vs baseline: 1.2926x; 1.0015x over previous
import functools

import jax
import jax.numpy as jnp
from jax import lax
from jax.experimental import pallas as pl
from jax.experimental.pallas import tpu as pltpu
from jax.experimental.pallas import tpu_sc as plsc

NC = 2
NS = 16
NW = NC * NS

CB = 4
NBUF = 2


def _gather_call(batch, seq, dim):
    bpw = batch // NW
    nch = bpw // CB
    s0 = (seq // 2 + 7) // 8 * 8

    mesh = plsc.VectorSubcoreMesh(core_axis_name="c", subcore_axis_name="s")

    @functools.partial(
        pl.kernel,
        out_type=jax.ShapeDtypeStruct((batch, seq, dim), jnp.float32),
        mesh=mesh,
        compiler_params=pltpu.CompilerParams(use_tc_tiling_on_sc=False),
        scratch_types=[
            pltpu.VMEM((bpw, seq), jnp.int32),
            pltpu.VMEM((NBUF, CB, seq, dim), jnp.float32),
            pltpu.SemaphoreType.DMA,
            pltpu.SemaphoreType.DMA,
            pltpu.SemaphoreType.DMA,
        ],
    )
    def gather_kernel(ids_hbm, table_hbm, out_hbm, idx_v, rows_v,
                      gsem0, gsem1, osem):
        gsems = (gsem0, gsem1)
        wid = lax.axis_index("s") * NC + lax.axis_index("c")
        b0 = wid * bpw
        pltpu.sync_copy(ids_hbm.at[pl.ds(b0, bpw)], idx_v)

        def issue_gathers(c, b, start):
            for br in range(CB):
                r = c * CB + br
                for (o, ln) in ((0, s0), (s0, seq - s0)):
                    src = table_hbm.at[idx_v.at[r, pl.ds(o, ln)]]
                    dst = rows_v.at[b, br].at[pl.ds(o, ln)]
                    if start:
                        pltpu.async_copy(src, dst, gsems[b])
                    else:
                        pltpu.make_async_copy(src, dst, gsems[b]).wait()

        def finish_chunk(c, b):
            issue_gathers(c, b, start=False)
            pltpu.async_copy(rows_v.at[b],
                             out_hbm.at[pl.ds(b0 + c * CB, CB)], osem)

        def chunk_body(c, _):
            for b in range(NBUF):
                @pl.when(lax.rem(c, NBUF) == b)
                def _(b=b):
                    @pl.when(c >= NBUF)
                    def _():
                        pltpu.make_async_copy(
                            rows_v.at[b],
                            out_hbm.at[pl.ds(b0 + (c - NBUF) * CB, CB)],
                            osem,
                        ).wait()
                    issue_gathers(c, b, start=True)
                    @pl.when(c >= 1)
                    def _():
                        finish_chunk(c - 1, 1 - b)
            return 0

        lax.fori_loop(0, nch, chunk_body, 0)
        finish_chunk(nch - 1, (nch - 1) % NBUF)
        for b in range(NBUF):
            c = nch - NBUF + b
            pltpu.make_async_copy(
                rows_v.at[c % NBUF],
                out_hbm.at[pl.ds(b0 + c * CB, CB)],
                osem,
            ).wait()

    return gather_kernel


def kernel(input_ids, base_table, special_A, special_B, lookup_A, lookup_B):
    del lookup_A, lookup_B
    batch, seq = input_ids.shape
    dim = base_table.shape[1]
    table = jnp.concatenate([base_table, special_A, special_B], axis=0)
    return _gather_call(batch, seq, dim)(input_ids, table)
